# baseline (speedup 1.0000x reference)
import jax
import jax.numpy as jnp
from jax.experimental import pallas as pl

_NB = 128


def _sgnroot(v):
    return v * jax.lax.rsqrt(jnp.maximum(jnp.abs(v), jnp.float32(1e-30)))


def _ktrans_block(x_ref, nbr_ref, w1t_ref, g1v_ref, b1v_ref, w2f_ref,
                  g2_ref, bb2_ref, wc_ref, bc_ref, out_ref):
    xb = x_ref[...]
    nb = nbr_ref[...]
    NB, D, F = nb.shape

    s = jnp.sum(nb, axis=1)
    fadj = xb[:, :, None] * s[:, None, :] + s[:, :, None] * xb[:, None, :]
    t = _sgnroot(fadj)
    den = jax.lax.dot_general(
        jnp.abs(t).reshape(NB * F, F), jnp.ones((F, 8), jnp.float32),
        (((1,), (0,)), ((), ())),
        preferred_element_type=jnp.float32)
    recip = 1.0 / (den[:, :1].reshape(NB, F, 1) + 1e-7)
    adj = t * jnp.swapaxes(recip, 1, 2)

    a1 = jax.lax.dot_general(
        adj.reshape(NB * F, F), w1t_ref[...], (((1,), (0,)), ((), ())),
        preferred_element_type=jnp.float32).reshape(NB, F, 64)
    nbx = jnp.concatenate([xb[:, None, :], nb], axis=1)
    n2x = jax.lax.dot_general(
        nbx, a1, (((2,), (1,)), ((0,), (0,))),
        preferred_element_type=jnp.float32)

    n2x = n2x * g1v_ref[...][None] + b1v_ref[...][None]
    n2x = n2x / (1.0 + jnp.abs(n2x))
    x1 = n2x[:, 0, :]
    n2s = jnp.sum(n2x[:, 1:, :], axis=1)

    x1t = x1.T.reshape(4, 16, NB)
    n2t = n2s.T.reshape(4, 16, NB)
    fadj2 = jnp.sum(
        x1t[:, :, None, :] * n2t[:, None, :, :]
        + n2t[:, :, None, :] * x1t[:, None, :, :], axis=0)
    t2 = _sgnroot(fadj2)
    recip2 = 1.0 / (jnp.sum(jnp.abs(t2), axis=0, keepdims=True) + 1e-7)
    adj2 = t2 * recip2

    xt2 = jnp.sum(x1t[:, :, None, :] * adj2[None, :, :, :], axis=1)

    x2 = jax.lax.dot_general(
        w2f_ref[...], xt2.reshape(64, NB), (((1,), (0,)), ((), ())),
        preferred_element_type=jnp.float32)
    x2 = x2 * g2_ref[...] + bb2_ref[...]
    x2 = x2 / (1.0 + jnp.abs(x2))

    out_ref[...] = jax.lax.dot_general(
        wc_ref[...], x2, (((1,), (0,)), ((), ())),
        preferred_element_type=jnp.float32) + bc_ref[...]


@jax.jit
def kernel(x, neighbor, W1, b1, gamma1, beta1, W2, b2, gamma2, beta2, Wc, bc):
    N, _, F = x.shape
    D = neighbor.shape[2]
    xb = x.reshape(N, F)
    nb = neighbor.reshape(N, D, F)

    inv = 1.0 / jnp.sqrt(1.0 + 1e-5)
    w1t = W1.reshape(64, F).T
    g1 = jnp.repeat(gamma1, 16)
    g1v = (g1 * inv)[None, :]
    b1v = (b1 * inv * g1 + jnp.repeat(beta1, 16))[None, :]

    w2f = W2.reshape(32, 64)
    g2 = (gamma2 * inv)[:, None]
    bb2 = (b2 * inv * gamma2 + beta2)[:, None]
    bcr = bc[:, None]

    grid = (pl.cdiv(N, _NB),)
    out_t = pl.pallas_call(
        _ktrans_block,
        grid=grid,
        in_specs=[
            pl.BlockSpec((_NB, F), lambda i: (i, 0)),
            pl.BlockSpec((_NB, D, F), lambda i: (i, 0, 0)),
            pl.BlockSpec((F, 64), lambda i: (0, 0)),
            pl.BlockSpec((1, 64), lambda i: (0, 0)),
            pl.BlockSpec((1, 64), lambda i: (0, 0)),
            pl.BlockSpec((32, 64), lambda i: (0, 0)),
            pl.BlockSpec((32, 1), lambda i: (0, 0)),
            pl.BlockSpec((32, 1), lambda i: (0, 0)),
            pl.BlockSpec((10, 32), lambda i: (0, 0)),
            pl.BlockSpec((10, 1), lambda i: (0, 0)),
        ],
        out_specs=pl.BlockSpec((10, _NB), lambda i: (0, i)),
        out_shape=jax.ShapeDtypeStruct((10, N), jnp.float32),
    )(xb, nb, w1t, g1v, b1v, w2f, g2, bb2, Wc, bcr)
    return out_t.T


# speedup vs baseline: 1.2775x; 1.2775x over previous
import jax
import jax.numpy as jnp
from jax.experimental import pallas as pl

_NB = 128


def _sgnroot(v):
    return v * jax.lax.rsqrt(jnp.maximum(jnp.abs(v), jnp.float32(1e-30)))


def _ktrans_block(x_ref, nbr_ref, w1t_ref, g1v_ref, b1v_ref, w2f_ref,
                  g2_ref, bb2_ref, wc_ref, bc_ref, out_ref):
    xb = x_ref[...]
    nb = nbr_ref[...]
    NB, D, F = nb.shape

    s = jnp.sum(nb, axis=1)
    fadj = xb[:, :, None] * s[:, None, :] + s[:, :, None] * xb[:, None, :]
    t = _sgnroot(fadj)
    recip = 1.0 / (jnp.sum(jnp.abs(t), axis=1, keepdims=True) + 1e-7)
    adj = t * recip

    a1 = jax.lax.dot_general(
        adj.reshape(NB * F, F), w1t_ref[...], (((1,), (0,)), ((), ())),
        preferred_element_type=jnp.float32).reshape(NB, F, 64)
    nbx = jnp.concatenate([xb[:, None, :], nb], axis=1)
    n2x = jax.lax.dot_general(
        nbx, a1, (((2,), (1,)), ((0,), (0,))),
        preferred_element_type=jnp.float32)

    n2x = n2x * g1v_ref[...][None] + b1v_ref[...][None]
    n2x = n2x / (1.0 + jnp.abs(n2x))
    x1 = n2x[:, 0, :]
    n2s = jnp.sum(n2x[:, 1:, :], axis=1)

    x1t = x1.T.reshape(4, 16, NB)
    n2t = n2s.T.reshape(4, 16, NB)
    fadj2 = jnp.sum(
        x1t[:, :, None, :] * n2t[:, None, :, :]
        + n2t[:, :, None, :] * x1t[:, None, :, :], axis=0)
    t2 = _sgnroot(fadj2)
    recip2 = 1.0 / (jnp.sum(jnp.abs(t2), axis=0, keepdims=True) + 1e-7)
    adj2 = t2 * recip2

    xt2 = jnp.sum(x1t[:, :, None, :] * adj2[None, :, :, :], axis=1)

    x2 = jax.lax.dot_general(
        w2f_ref[...], xt2.reshape(64, NB), (((1,), (0,)), ((), ())),
        preferred_element_type=jnp.float32)
    x2 = x2 * g2_ref[...] + bb2_ref[...]
    x2 = x2 / (1.0 + jnp.abs(x2))

    out_ref[...] = jax.lax.dot_general(
        x2.T, wc_ref[...], (((1,), (0,)), ((), ())),
        preferred_element_type=jnp.float32) + bc_ref[...]


@jax.jit
def kernel(x, neighbor, W1, b1, gamma1, beta1, W2, b2, gamma2, beta2, Wc, bc):
    N, _, F = x.shape
    D = neighbor.shape[2]
    xb = x.reshape(N, F)
    nb = neighbor.reshape(N, D, F)

    inv = 1.0 / jnp.sqrt(1.0 + 1e-5)
    w1t = W1.reshape(64, F).T
    g1 = jnp.repeat(gamma1, 16)
    g1v = (g1 * inv)[None, :]
    b1v = (b1 * inv * g1 + jnp.repeat(beta1, 16))[None, :]

    w2f = W2.reshape(32, 64)
    g2 = (gamma2 * inv)[:, None]
    bb2 = (b2 * inv * gamma2 + beta2)[:, None]
    bcr = bc[:, None]

    grid = (pl.cdiv(N, _NB),)
    out_t = pl.pallas_call(
        _ktrans_block,
        grid=grid,
        in_specs=[
            pl.BlockSpec((_NB, F), lambda i: (i, 0)),
            pl.BlockSpec((_NB, D, F), lambda i: (i, 0, 0)),
            pl.BlockSpec((F, 64), lambda i: (0, 0)),
            pl.BlockSpec((1, 64), lambda i: (0, 0)),
            pl.BlockSpec((1, 64), lambda i: (0, 0)),
            pl.BlockSpec((32, 64), lambda i: (0, 0)),
            pl.BlockSpec((32, 1), lambda i: (0, 0)),
            pl.BlockSpec((32, 1), lambda i: (0, 0)),
            pl.BlockSpec((32, 10), lambda i: (0, 0)),
            pl.BlockSpec((1, 10), lambda i: (0, 0)),
        ],
        out_specs=pl.BlockSpec((_NB, 10), lambda i: (i, 0)),
        out_shape=jax.ShapeDtypeStruct((N, 10), jnp.float32),
    )(xb, nb, w1t, g1v, b1v, w2f, g2, bb2, Wc.T, bc[None, :])
    return out_t


# speedup vs baseline: 1.3105x; 1.0259x over previous
import jax
import jax.numpy as jnp
from jax.experimental import pallas as pl

_NB = 128


def _sgnroot(v):
    return v * jax.lax.rsqrt(jnp.maximum(jnp.abs(v), jnp.float32(1e-30)))


def _ktrans_block(x_ref, nbr_ref, w1t_ref, g1v_ref, b1v_ref, w2f_ref,
                  g2_ref, bb2_ref, wc_ref, bc_ref, out_ref):
    xb = x_ref[...]
    nb = nbr_ref[...]
    NB, D, F = nb.shape

    s = jnp.sum(nb, axis=1)
    fadj = xb[:, :, None] * s[:, None, :] + s[:, :, None] * xb[:, None, :]
    t = _sgnroot(fadj)
    recip = 1.0 / (jnp.sum(jnp.abs(t), axis=1, keepdims=True) + 1e-7)
    adj = t * recip

    a1 = jax.lax.dot_general(
        adj.reshape(NB * F, F), w1t_ref[...], (((1,), (0,)), ((), ())),
        preferred_element_type=jnp.float32).reshape(NB, F, 64)
    nbx = jnp.concatenate([xb[:, None, :], nb], axis=1)
    n2x = jax.lax.dot_general(
        nbx, a1, (((2,), (1,)), ((0,), (0,))),
        preferred_element_type=jnp.float32)

    n2x = n2x * g1v_ref[...][None] + b1v_ref[...][None]
    n2x = n2x / (1.0 + jnp.abs(n2x))
    x1 = n2x[:, 0, :]
    n2s = jnp.sum(n2x[:, 1:, :], axis=1)

    x1t = x1.T.reshape(4, 16, NB)
    n2t = n2s.T.reshape(4, 16, NB)
    fadj2 = jnp.sum(
        x1t[:, :, None, :] * n2t[:, None, :, :]
        + n2t[:, :, None, :] * x1t[:, None, :, :], axis=0)
    t2 = _sgnroot(fadj2)
    recip2 = 1.0 / (jnp.sum(jnp.abs(t2), axis=0, keepdims=True) + 1e-7)
    adj2 = t2 * recip2

    xt2 = jnp.sum(x1t[:, :, None, :] * adj2[None, :, :, :], axis=1)

    x2 = jax.lax.dot_general(
        w2f_ref[...], xt2.reshape(64, NB), (((1,), (0,)), ((), ())),
        preferred_element_type=jnp.float32)
    x2 = x2 * g2_ref[...] + bb2_ref[...]
    x2 = x2 / (1.0 + jnp.abs(x2))

    out_ref[...] = jax.lax.dot_general(
        wc_ref[...], x2, (((1,), (0,)), ((), ())),
        preferred_element_type=jnp.float32) + bc_ref[...]


@jax.jit
def kernel(x, neighbor, W1, b1, gamma1, beta1, W2, b2, gamma2, beta2, Wc, bc):
    N, _, F = x.shape
    D = neighbor.shape[2]
    xb = x.reshape(N, F)
    nb = neighbor.reshape(N, D, F)

    inv = 1.0 / jnp.sqrt(1.0 + 1e-5)
    w1t = W1.reshape(64, F).T
    g1 = jnp.repeat(gamma1, 16)
    g1v = (g1 * inv)[None, :]
    b1v = (b1 * inv * g1 + jnp.repeat(beta1, 16))[None, :]

    w2f = W2.reshape(32, 64)
    g2 = (gamma2 * inv)[:, None]
    bb2 = (b2 * inv * gamma2 + beta2)[:, None]
    bcr = bc[:, None]

    grid = (pl.cdiv(N, _NB),)
    out_t = pl.pallas_call(
        _ktrans_block,
        grid=grid,
        in_specs=[
            pl.BlockSpec((_NB, F), lambda i: (i, 0)),
            pl.BlockSpec((_NB, D, F), lambda i: (i, 0, 0)),
            pl.BlockSpec((F, 64), lambda i: (0, 0)),
            pl.BlockSpec((1, 64), lambda i: (0, 0)),
            pl.BlockSpec((1, 64), lambda i: (0, 0)),
            pl.BlockSpec((32, 64), lambda i: (0, 0)),
            pl.BlockSpec((32, 1), lambda i: (0, 0)),
            pl.BlockSpec((32, 1), lambda i: (0, 0)),
            pl.BlockSpec((10, 32), lambda i: (0, 0)),
            pl.BlockSpec((10, 1), lambda i: (0, 0)),
        ],
        out_specs=pl.BlockSpec((10, _NB), lambda i: (0, i)),
        out_shape=jax.ShapeDtypeStruct((10, N), jnp.float32),
    )(xb, nb, w1t, g1v, b1v, w2f, g2, bb2, Wc, bc[:, None])
    return out_t.T
